# Initial kernel scaffold; baseline (speedup 1.0000x reference)
import jax
import jax.numpy as jnp
from jax.experimental import pallas as pl


def kernel(x, edge_index, W, b):
    raise NotImplementedError("write your pallas kernel here")


# speedup vs baseline: 20.9501x; 20.9501x over previous
import functools

import jax
import jax.numpy as jnp
from jax import lax
from jax.experimental import pallas as pl
from jax.experimental.pallas import tpu as pltpu
from jax.experimental.pallas import tpu_sc as plsc

_NC, _NS = 2, 16
_N, _D, _E = 10000, 256, 160000
_H = _D // 2
_NPR = 10240
_SEG = _NPR // _NS

_A_CH = 125
_A_NJ = _E // (_NC * _NS) // _A_CH
_C_CH = 100
_C_NJ = _E // _NS // _C_CH

_mesh = plsc.VectorSubcoreMesh(
    core_axis_name="c", subcore_axis_name="s", num_cores=_NC, num_subcores=_NS
)


_DW = 128


@functools.partial(
    pl.kernel,
    out_type=jax.ShapeDtypeStruct((_NC, _NPR, _DW), jnp.float32),
    mesh=_mesh,
    scratch_types=[
        pltpu.VMEM((_A_NJ, _A_CH), jnp.int32),
        pltpu.VMEM((_A_CH, _DW), jnp.float32),
        [pltpu.SemaphoreType.DMA] * 4,
        pltpu.VMEM_SHARED((_NPR, _DW), jnp.float32),
    ],
)
def _deg_call(col3, zseg, ones, degs, cidx_v, ones_v, sems, deg_sp):
    c = lax.axis_index("c")
    s = lax.axis_index("s")
    w = c * _NS + s
    r0 = s * _SEG
    pltpu.sync_copy(zseg, deg_sp.at[pl.ds(r0, _SEG)])
    pltpu.sync_copy(col3.at[w], cidx_v)
    pltpu.sync_copy(ones, ones_v)
    plsc.subcore_barrier()

    for m in range(4):
        pltpu.async_copy(ones_v, deg_sp.at[cidx_v.at[m]], sems[m], add=True)

    def quad(p, carry):
        for m in range(4):
            j = 4 * p + m
            pltpu.make_async_copy(ones_v, deg_sp.at[cidx_v.at[j]], sems[m]).wait()
            pltpu.async_copy(ones_v, deg_sp.at[cidx_v.at[j + 4]], sems[m], add=True)
        return carry

    lax.fori_loop(0, _A_NJ // 4 - 1, quad, None)
    for m in range(4):
        j = _A_NJ - 4 + m
        pltpu.make_async_copy(ones_v, deg_sp.at[cidx_v.at[j]], sems[m]).wait()
    plsc.subcore_barrier()
    pltpu.sync_copy(deg_sp.at[pl.ds(r0, _SEG)], degs.at[c, pl.ds(r0, _SEG)])


_RB = 1000


def _mm_body(x_ref, w_ref, d_ref, hs0_ref, hs1_ref, dv_ref):
    h = jnp.dot(x_ref[...], w_ref[...], preferred_element_type=jnp.float32)
    deg = d_ref[0, :, :1] + d_ref[1, :, :1] + 1.0
    dinv = lax.rsqrt(deg)
    hs = h * dinv
    hs0_ref[...] = hs[:, :_H]
    hs1_ref[...] = hs[:, _H:]
    dv_ref[...] = jnp.broadcast_to(dinv, (_RB, _DW))


_mm_call = pl.pallas_call(
    _mm_body,
    grid=(_N // _RB,),
    in_specs=[
        pl.BlockSpec((_RB, _D), lambda i: (i, 0)),
        pl.BlockSpec((_D, _D), lambda i: (0, 0)),
        pl.BlockSpec((_NC, _RB, _DW), lambda i: (0, i, 0)),
    ],
    out_specs=[
        pl.BlockSpec((_RB, _H), lambda i: (i, 0)),
        pl.BlockSpec((_RB, _H), lambda i: (i, 0)),
        pl.BlockSpec((_RB, _DW), lambda i: (i, 0)),
    ],
    out_shape=[
        jax.ShapeDtypeStruct((_NPR, _H), jnp.float32),
        jax.ShapeDtypeStruct((_NPR, _H), jnp.float32),
        jax.ShapeDtypeStruct((_NPR, _DW), jnp.float32),
    ],
)


@functools.partial(
    pl.kernel,
    out_type=[
        jax.ShapeDtypeStruct((_NPR, _H), jnp.float32),
        jax.ShapeDtypeStruct((_NPR, _H), jnp.float32),
    ],
    mesh=_mesh,
    scratch_types=[
        [pltpu.VMEM((2, _C_CH), jnp.int32)] * 6,
        [pltpu.VMEM((_C_CH, _H), jnp.float32)] * 3,
        [pltpu.SemaphoreType.DMA] * 6,
        [pltpu.SemaphoreType.DMA] * 3,
        [pltpu.SemaphoreType.DMA] * 3,
        pltpu.VMEM_SHARED((_NPR, _H), jnp.float32),
    ],
)
def _agg_call(hs0, hs1, ec4, s0, s1, idxs, rows, semi, semg, sems, agg_sp):
    c = lax.axis_index("c")
    s = lax.axis_index("s")

    def half(hs_ref, out_ref):
        r0 = s * _SEG
        pltpu.sync_copy(hs_ref.at[pl.ds(r0, _SEG)], agg_sp.at[pl.ds(r0, _SEG)])
        plsc.subcore_barrier()

        def chunk_body(j, m, first, do_idx, do_gather):
            b = m % 3
            pltpu.make_async_copy(hs_ref.at[idxs[m].at[0]], rows[b], semg[b]).wait()
            pltpu.async_copy(rows[b], agg_sp.at[idxs[m].at[1]], sems[b], add=True)
            if do_idx:
                m3 = (m + 3) % 6
                pltpu.async_copy(ec4.at[s, j + 3], idxs[m3], semi[m3])
            if not first:
                b1 = (m + 2) % 3
                pltpu.make_async_copy(
                    rows[b1], agg_sp.at[idxs[(m + 5) % 6].at[1]], sems[b1]
                ).wait()
            if do_gather:
                m2 = (m + 2) % 6
                b2 = (m + 2) % 3
                pltpu.make_async_copy(ec4.at[s, j + 2], idxs[m2], semi[m2]).wait()
                pltpu.async_copy(hs_ref.at[idxs[m2].at[0]], rows[b2], semg[b2])

        for m in range(2):
            pltpu.sync_copy(ec4.at[s, m], idxs[m])
        pltpu.async_copy(ec4.at[s, 2], idxs[2], semi[2])
        for m in range(2):
            pltpu.async_copy(hs_ref.at[idxs[m].at[0]], rows[m], semg[m])
        for j in range(6):
            chunk_body(j, j, first=(j == 0), do_idx=True, do_gather=True)

        def sextet(p, carry):
            for m in range(6):
                chunk_body(6 * p + m, m, first=False, do_idx=True, do_gather=True)
            return carry

        lax.fori_loop(1, _C_NJ // 6, sextet, None)
        for j in range(_C_NJ - 4, _C_NJ):
            m = j % 6
            chunk_body(
                j, m, first=False,
                do_idx=(j + 3 < _C_NJ), do_gather=(j + 2 < _C_NJ),
            )
        jl = _C_NJ - 1
        pltpu.make_async_copy(
            rows[jl % 3], agg_sp.at[idxs[jl % 6].at[1]], sems[jl % 3]
        ).wait()
        plsc.subcore_barrier()
        pltpu.sync_copy(agg_sp.at[pl.ds(r0, _SEG)], out_ref.at[pl.ds(r0, _SEG)])

    @pl.when(c == 0)
    def _():
        half(hs0, s0)

    @pl.when(c == 1)
    def _():
        half(hs1, s1)


def _out_body(x_ref, s0_ref, s1_ref, dv_ref, b_ref, out_ref):
    dinv = dv_ref[:, :1]
    out_ref[:, :_H] = x_ref[:, :_H] + jnp.maximum(
        dinv * s0_ref[...] + b_ref[0, :_H][None, :], 0.0
    )
    out_ref[:, _H:] = x_ref[:, _H:] + jnp.maximum(
        dinv * s1_ref[...] + b_ref[0, _H:][None, :], 0.0
    )


_out_call = pl.pallas_call(
    _out_body,
    grid=(_N // _RB,),
    in_specs=[
        pl.BlockSpec((_RB, _D), lambda i: (i, 0)),
        pl.BlockSpec((_RB, _H), lambda i: (i, 0)),
        pl.BlockSpec((_RB, _H), lambda i: (i, 0)),
        pl.BlockSpec((_RB, _DW), lambda i: (i, 0)),
        pl.BlockSpec((1, _D), lambda i: (0, 0)),
    ],
    out_specs=pl.BlockSpec((_RB, _D), lambda i: (i, 0)),
    out_shape=jax.ShapeDtypeStruct((_N, _D), jnp.float32),
)


def kernel(x, edge_index, W, b):
    row = edge_index[0]
    col = edge_index[1]
    col3 = col.reshape(_NC * _NS, _A_NJ, _A_CH)
    zseg = jnp.zeros((_SEG, _DW), jnp.float32)
    ones = jnp.ones((_A_CH, _DW), jnp.float32)
    degs = _deg_call(col3, zseg, ones)
    hs0, hs1, dv = _mm_call(x, W, degs)
    ec4 = jnp.stack(
        [row.reshape(_NS, _C_NJ, _C_CH), col.reshape(_NS, _C_NJ, _C_CH)], axis=2
    )
    s0, s1 = _agg_call(hs0, hs1, ec4)
    return _out_call(x, s0, s1, dv, b.reshape(1, _D))


# speedup vs baseline: 21.0248x; 1.0036x over previous
import functools

import jax
import jax.numpy as jnp
from jax import lax
from jax.experimental import pallas as pl
from jax.experimental.pallas import tpu as pltpu
from jax.experimental.pallas import tpu_sc as plsc

_NC, _NS = 2, 16
_N, _D, _E = 10000, 256, 160000
_H = _D // 2
_NPR = 10240
_SEG = _NPR // _NS

_A_CH = 125
_A_NJ = _E // (_NC * _NS) // _A_CH
_C_CH = 100
_C_NJ = _E // _NS // _C_CH

_mesh = plsc.VectorSubcoreMesh(
    core_axis_name="c", subcore_axis_name="s", num_cores=_NC, num_subcores=_NS
)


_DW = 128


@functools.partial(
    pl.kernel,
    out_type=jax.ShapeDtypeStruct((_NC, _NPR, _DW), jnp.float32),
    mesh=_mesh,
    scratch_types=[
        pltpu.VMEM((_A_NJ, _A_CH), jnp.int32),
        pltpu.VMEM((_A_CH, _DW), jnp.float32),
        [pltpu.SemaphoreType.DMA] * 4,
        pltpu.VMEM_SHARED((_NPR, _DW), jnp.float32),
    ],
)
def _deg_call(col3, zseg, ones, degs, cidx_v, ones_v, sems, deg_sp):
    c = lax.axis_index("c")
    s = lax.axis_index("s")
    w = c * _NS + s
    r0 = s * _SEG
    pltpu.sync_copy(zseg, deg_sp.at[pl.ds(r0, _SEG)])
    pltpu.sync_copy(col3.at[w], cidx_v)
    pltpu.sync_copy(ones, ones_v)
    plsc.subcore_barrier()

    for m in range(4):
        pltpu.async_copy(ones_v, deg_sp.at[cidx_v.at[m]], sems[m], add=True)

    def quad(p, carry):
        for m in range(4):
            j = 4 * p + m
            pltpu.make_async_copy(ones_v, deg_sp.at[cidx_v.at[j]], sems[m]).wait()
            pltpu.async_copy(ones_v, deg_sp.at[cidx_v.at[j + 4]], sems[m], add=True)
        return carry

    lax.fori_loop(0, _A_NJ // 4 - 1, quad, None)
    for m in range(4):
        j = _A_NJ - 4 + m
        pltpu.make_async_copy(ones_v, deg_sp.at[cidx_v.at[j]], sems[m]).wait()
    plsc.subcore_barrier()
    pltpu.sync_copy(deg_sp.at[pl.ds(r0, _SEG)], degs.at[c, pl.ds(r0, _SEG)])


_RB = 1000


def _mm_body(x_ref, w_ref, d_ref, hs0_ref, hs1_ref):
    h = jnp.dot(x_ref[...], w_ref[...], preferred_element_type=jnp.float32)
    deg = d_ref[0, :, :1] + d_ref[1, :, :1] + 1.0
    hs = h * lax.rsqrt(deg)
    hs0_ref[...] = hs[:, :_H]
    hs1_ref[...] = hs[:, _H:]


_mm_call = pl.pallas_call(
    _mm_body,
    grid=(_N // _RB,),
    in_specs=[
        pl.BlockSpec((_RB, _D), lambda i: (i, 0)),
        pl.BlockSpec((_D, _D), lambda i: (0, 0)),
        pl.BlockSpec((_NC, _RB, _DW), lambda i: (0, i, 0)),
    ],
    out_specs=[
        pl.BlockSpec((_RB, _H), lambda i: (i, 0)),
        pl.BlockSpec((_RB, _H), lambda i: (i, 0)),
    ],
    out_shape=[
        jax.ShapeDtypeStruct((_NPR, _H), jnp.float32),
        jax.ShapeDtypeStruct((_NPR, _H), jnp.float32),
    ],
)


@functools.partial(
    pl.kernel,
    out_type=[
        jax.ShapeDtypeStruct((_NPR, _H), jnp.float32),
        jax.ShapeDtypeStruct((_NPR, _H), jnp.float32),
    ],
    mesh=_mesh,
    scratch_types=[
        [pltpu.VMEM((2, _C_CH), jnp.int32)] * 6,
        [pltpu.VMEM((_C_CH, _H), jnp.float32)] * 3,
        [pltpu.SemaphoreType.DMA] * 6,
        [pltpu.SemaphoreType.DMA] * 3,
        [pltpu.SemaphoreType.DMA] * 3,
        pltpu.VMEM_SHARED((_NPR, _H), jnp.float32),
    ],
)
def _agg_call(hs0, hs1, ec4, s0, s1, idxs, rows, semi, semg, sems, agg_sp):
    c = lax.axis_index("c")
    s = lax.axis_index("s")

    def half(hs_ref, out_ref):
        r0 = s * _SEG
        pltpu.sync_copy(hs_ref.at[pl.ds(r0, _SEG)], agg_sp.at[pl.ds(r0, _SEG)])
        plsc.subcore_barrier()

        def chunk_body(j, m, first, do_idx, do_gather):
            b = m % 3
            pltpu.make_async_copy(hs_ref.at[idxs[m].at[0]], rows[b], semg[b]).wait()
            pltpu.async_copy(rows[b], agg_sp.at[idxs[m].at[1]], sems[b], add=True)
            if do_idx:
                m3 = (m + 3) % 6
                pltpu.async_copy(ec4.at[s, j + 3], idxs[m3], semi[m3])
            if not first:
                b1 = (m + 2) % 3
                pltpu.make_async_copy(
                    rows[b1], agg_sp.at[idxs[(m + 5) % 6].at[1]], sems[b1]
                ).wait()
            if do_gather:
                m2 = (m + 2) % 6
                b2 = (m + 2) % 3
                pltpu.make_async_copy(ec4.at[s, j + 2], idxs[m2], semi[m2]).wait()
                pltpu.async_copy(hs_ref.at[idxs[m2].at[0]], rows[b2], semg[b2])

        for m in range(2):
            pltpu.sync_copy(ec4.at[s, m], idxs[m])
        pltpu.async_copy(ec4.at[s, 2], idxs[2], semi[2])
        for m in range(2):
            pltpu.async_copy(hs_ref.at[idxs[m].at[0]], rows[m], semg[m])
        for j in range(6):
            chunk_body(j, j, first=(j == 0), do_idx=True, do_gather=True)

        def sextet(p, carry):
            for m in range(6):
                chunk_body(6 * p + m, m, first=False, do_idx=True, do_gather=True)
            return carry

        lax.fori_loop(1, _C_NJ // 6, sextet, None)
        for j in range(_C_NJ - 4, _C_NJ):
            m = j % 6
            chunk_body(
                j, m, first=False,
                do_idx=(j + 3 < _C_NJ), do_gather=(j + 2 < _C_NJ),
            )
        jl = _C_NJ - 1
        pltpu.make_async_copy(
            rows[jl % 3], agg_sp.at[idxs[jl % 6].at[1]], sems[jl % 3]
        ).wait()
        plsc.subcore_barrier()
        pltpu.sync_copy(agg_sp.at[pl.ds(r0, _SEG)], out_ref.at[pl.ds(r0, _SEG)])

    @pl.when(c == 0)
    def _():
        half(hs0, s0)

    @pl.when(c == 1)
    def _():
        half(hs1, s1)


def _out_body(x_ref, s0_ref, s1_ref, d_ref, b_ref, out_ref):
    deg = d_ref[0, :, :1] + d_ref[1, :, :1] + 1.0
    dinv = lax.rsqrt(deg)
    out_ref[:, :_H] = x_ref[:, :_H] + jnp.maximum(
        dinv * s0_ref[...] + b_ref[0, :_H][None, :], 0.0
    )
    out_ref[:, _H:] = x_ref[:, _H:] + jnp.maximum(
        dinv * s1_ref[...] + b_ref[0, _H:][None, :], 0.0
    )


_out_call = pl.pallas_call(
    _out_body,
    grid=(_N // _RB,),
    in_specs=[
        pl.BlockSpec((_RB, _D), lambda i: (i, 0)),
        pl.BlockSpec((_RB, _H), lambda i: (i, 0)),
        pl.BlockSpec((_RB, _H), lambda i: (i, 0)),
        pl.BlockSpec((_NC, _RB, _DW), lambda i: (0, i, 0)),
        pl.BlockSpec((1, _D), lambda i: (0, 0)),
    ],
    out_specs=pl.BlockSpec((_RB, _D), lambda i: (i, 0)),
    out_shape=jax.ShapeDtypeStruct((_N, _D), jnp.float32),
)


def kernel(x, edge_index, W, b):
    row = edge_index[0]
    col = edge_index[1]
    col3 = col.reshape(_NC * _NS, _A_NJ, _A_CH)
    zseg = jnp.zeros((_SEG, _DW), jnp.float32)
    ones = jnp.ones((_A_CH, _DW), jnp.float32)
    degs = _deg_call(col3, zseg, ones)
    hs0, hs1 = _mm_call(x, W, degs)
    ec4 = jnp.stack(
        [row.reshape(_NS, _C_NJ, _C_CH), col.reshape(_NS, _C_NJ, _C_CH)], axis=2
    )
    s0, s1 = _agg_call(hs0, hs1, ec4)
    return _out_call(x, s0, s1, degs, b.reshape(1, _D))
